# baseline (speedup 1.0000x reference)
import jax
import jax.numpy as jnp
from jax import lax
from jax.experimental import pallas as pl
from jax.experimental.pallas import tpu as pltpu
from jax.experimental.pallas import tpu_sc as plsc

_N_TABLES = 26
_VOCAB = 100000
_WIDTH = 32
_BATCH = 4096

_NC = 2
_NS = 16
_L = 16
_NW = _NC * _NS
_BPW = _BATCH // _NW
_PACK = 128 // _WIDTH


def _body(tab_hbm, x_hbm, out_hbm, xraw_v, gidx_v, buf, stage, sem):
    wid = lax.axis_index("s") * _NC + lax.axis_index("c")
    base = wid * _BPW

    pltpu.sync_copy(x_hbm.at[pl.ds(base, _BPW)], xraw_v)

    for j in range(_BPW // _L):
        sl = pl.ds(j * _L, _L)
        gidx_v[sl] = xraw_v[sl] >> 2

    def fire(t, slot):
        pltpu.async_copy(tab_hbm.at[t].at[gidx_v], buf.at[slot], sem)

    def wait(t, slot):
        pltpu.make_async_copy(tab_hbm.at[t].at[gidx_v], buf.at[slot], sem).wait()

    fire(0, 0)

    def step(t, carry):
        slot = lax.rem(t, 2)

        @pl.when(t + 1 < _N_TABLES)
        def _():
            fire(t + 1, lax.rem(t + 1, 2))

        wait(t, slot)

        def extract(g, c):
            xv = xraw_v[pl.ds(g * _L, _L)]
            for j in range(_L):
                off = (xv[j] & 3) * _WIDTH
                q = g * (_L // _PACK) + (j // _PACK)
                r = (j % _PACK) * _WIDTH
                b = g * _L + j
                stage[q, pl.ds(r, _L)] = buf[slot, b, pl.ds(off, _L)]
                stage[q, pl.ds(r + _L, _L)] = buf[slot, b, pl.ds(off + _L, _L)]
            return c

        lax.fori_loop(0, _BPW // _L, extract, 0)

        pltpu.sync_copy(stage, out_hbm.at[t, pl.ds(wid * (_BPW // _PACK), _BPW // _PACK), :])
        return carry

    lax.fori_loop(0, _N_TABLES, step, 0)


@jax.jit
def _lookup(tables, x):
    tables128 = tables.reshape(_N_TABLES, _VOCAB // _PACK, 128)
    mesh = plsc.VectorSubcoreMesh(core_axis_name="c", subcore_axis_name="s")
    out128 = pl.kernel(
        _body,
        out_type=jax.ShapeDtypeStruct((_N_TABLES, _BATCH // _PACK, 128), jnp.float32),
        mesh=mesh,
        scratch_types=[
            pltpu.VMEM((_BPW,), jnp.int32),
            pltpu.VMEM((_BPW,), jnp.int32),
            pltpu.VMEM((2, _BPW, 128), jnp.float32),
            pltpu.VMEM((_BPW // _PACK, 128), jnp.float32),
            pltpu.SemaphoreType.DMA,
        ],
    )(tables128, x)
    return out128.reshape(_N_TABLES, _BATCH, _WIDTH)


def kernel(x, tables):
    return _lookup(tables, x.astype(jnp.int32))


# speedup vs baseline: 3.1302x; 3.1302x over previous
import jax
import jax.numpy as jnp
from jax import lax
from jax.experimental import pallas as pl
from jax.experimental.pallas import tpu as pltpu
from jax.experimental.pallas import tpu_sc as plsc

_NT = 26
_V = 100000
_W = 32
_B = 4096

_NC = 2
_NS = 16
_L = 16

_TPC = _NT // _NC
_P0 = 7
_CHUNK = 1024
_VMAIN = (_V // 128) * 128
_VTAIL = _V - _VMAIN
_NCH = -(-_VMAIN // _CHUNK)
_TAILC = _VMAIN - (_NCH - 1) * _CHUNK
_WPT = _B * _W
_SPW = _P0 * _WPT
_DUMP = _SPW


def _body(tab_hbm, x_hbm, tail_hbm, out_hbm, xv, blk, tlv, ml, sdat, sidx, spm):
    c = lax.axis_index("c")
    s = lax.axis_index("s")

    pltpu.sync_copy(x_hbm, xv)
    iota = lax.iota(jnp.int32, _L)

    def scan(lo, hi):
        def scan_g(g, cnt):
            xg = xv[pl.ds(g * _L, _L)]
            m = (xg >= lo) & (xg < hi)
            plsc.store_compressed(
                ml.at[pl.ds(cnt, _L)], (xg << 12) | (iota + g * _L), mask=m
            )
            pc = plsc.all_reduce_population_count(m)
            return cnt + pc[0]

        return lax.fori_loop(0, _B // _L, scan_g, 0)

    def extract(tl, mg, k, lo, gather_col):
        pk = ml[pl.ds(mg * _L, _L)]
        bm = pk & (_B - 1)
        vrel = ((pk >> 12) - lo) & (_CHUNK - 1)
        padm = (iota + mg * _L) < k
        base = tl * _WPT + (bm >> 7) * 1024 + (bm & 127)
        for w in range(_W):
            rp = (w // 8) * 32768 + (w % 8) * 128
            data = gather_col(w, vrel)
            idxw = jnp.where(padm, base + rp, _DUMP)
            row = w // 8
            sl = pl.ds((w % 8) * _L, _L)
            sdat[row, sl] = data
            sidx[row, sl] = idxw
        for row in range(4):
            pltpu.sync_copy(sdat.at[row], spm.at[sidx.at[row]])

    def phase(t0, ntp):
        def process(cid, size):
            lo = cid * _CHUNK
            k = scan(lo, lo + size)
            ng = (k + _L - 1) // _L

            def per_table(tl, carry):
                t = c * _TPC + t0 + tl
                pltpu.sync_copy(
                    tab_hbm.at[t, :, pl.ds(lo, size)], blk.at[:, pl.ds(0, size)]
                )

                def gather_col(w, vrel):
                    wv = jnp.full((_L,), w, jnp.int32)
                    return plsc.load_gather(blk, [wv, vrel])

                def per_mg(mg, carry2):
                    extract(tl, mg, k, lo, gather_col)
                    return carry2

                lax.fori_loop(0, ng, per_mg, 0)
                return carry

            lax.fori_loop(0, ntp, per_table, 0)

        def chunk_iter(i, carry):
            cid = s + i * _NS

            @pl.when(cid < _NCH - 1)
            def _():
                process(cid, _CHUNK)

            @pl.when(cid == _NCH - 1)
            def _():
                process(cid, _TAILC)

            return carry

        lax.fori_loop(0, -(-_NCH // _NS), chunk_iter, 0)

        @pl.when(s == _NS - 1)
        def _():
            k = scan(_VMAIN, _V)
            ng = (k + _L - 1) // _L

            def per_table(tl, carry):
                t = c * _TPC + t0 + tl
                pltpu.sync_copy(
                    tail_hbm.at[pl.ds(t * (_VTAIL * _W), _VTAIL * _W)], tlv
                )

                def gather_col(w, vrel):
                    flat = (vrel & (_VTAIL - 1)) * _W + w
                    return plsc.load_gather(tlv, [flat])

                def per_mg(mg, carry2):
                    extract(tl, mg, k, _VMAIN, gather_col)
                    return carry2

                lax.fori_loop(0, ng, per_mg, 0)
                return carry

            lax.fori_loop(0, ntp, per_table, 0)

        plsc.subcore_barrier()

        per_w = ntp * _WPT // _NS
        off = s * per_w
        pltpu.sync_copy(
            spm.at[pl.ds(off, per_w)],
            out_hbm.at[pl.ds((c * _TPC + t0) * _WPT + off, per_w)],
        )
        plsc.subcore_barrier()

    phase(0, _P0)
    phase(_P0, _TPC - _P0)


@jax.jit
def _lookup(tables, x):
    tab_t = jnp.transpose(tables, (0, 2, 1))
    tail = tables[:, _VMAIN:, :].reshape(_NT * _VTAIL * _W)
    mesh = plsc.VectorSubcoreMesh(core_axis_name="c", subcore_axis_name="s")
    out1d = pl.kernel(
        _body,
        compiler_params=pltpu.CompilerParams(needs_layout_passes=False),
        out_type=jax.ShapeDtypeStruct((_NT * _WPT,), jnp.float32),
        mesh=mesh,
        scratch_types=[
            pltpu.VMEM((_B,), jnp.int32),
            pltpu.VMEM((_W, _CHUNK), jnp.float32),
            pltpu.VMEM((_VTAIL * _W,), jnp.float32),
            pltpu.VMEM((_B + _L,), jnp.int32),
            pltpu.VMEM((4, 128), jnp.float32),
            pltpu.VMEM((4, 128), jnp.int32),
            pltpu.VMEM_SHARED((_SPW + _L,), jnp.float32),
        ],
    )(tab_t, x, tail)
    out = out1d.reshape(_NT, 4, 32, 8, 128)
    return out.transpose(0, 2, 4, 1, 3).reshape(_NT, _B, _W)


def kernel(x, tables):
    return _lookup(tables, x.astype(jnp.int32))


# speedup vs baseline: 4.1134x; 1.3141x over previous
import jax
import jax.numpy as jnp
from jax import lax
from jax.experimental import pallas as pl
from jax.experimental.pallas import tpu as pltpu
from jax.experimental.pallas import tpu_sc as plsc

_NT = 26
_V = 100000
_W = 32
_B = 4096

_NC = 2
_NS = 16
_L = 16

_TPC = _NT // _NC
_P0 = 7
_CHUNK = 512
_CSH = 9
_VMAIN = (_V // 128) * 128
_VTAIL = _V - _VMAIN
_NCH = -(-_VMAIN // _CHUNK)
_TAILC = _VMAIN - (_NCH - 1) * _CHUNK
_WPT = _B * _W
_SPW = _P0 * _WPT
_DUMP = _SPW
_PAD = 3 * 32768 + 7 * 128 + _L + 16
_SENT = 1 << 30


def _body(tab_hbm, x_hbm, tail_hbm, out_hbm, xbuf, blk, tlv, ml, lvl1,
          sdat, sidx, spm, sem_b, sem_s):
    c = lax.axis_index("c")
    s = lax.axis_index("s")
    iota = lax.iota(jnp.int32, _L)

    def l1_outer(si, cnt):
        pltpu.sync_copy(x_hbm.at[pl.ds(si * 1024, 1024)], xbuf)

        def l1_g(gi, cnt2):
            xg = xbuf[pl.ds(gi * _L, _L)]
            m = ((xg >> _CSH) & (_NS - 1)) == s
            plsc.store_compressed(
                ml_dst(lvl1, cnt2), (xg << 12) | (iota + si * 1024 + gi * _L),
                mask=m,
            )
            pc = plsc.all_reduce_population_count(m)
            return cnt2 + pc[0]

        return lax.fori_loop(0, 1024 // _L, l1_g, cnt)

    def ml_dst(ref, cnt):
        return ref.at[pl.ds(cnt, _L)]

    k1 = lax.fori_loop(0, _B // 1024, l1_outer, 0)
    lvl1[pl.ds(k1, _L)] = jnp.full((_L,), _SENT, jnp.int32)
    ng1 = (k1 + _L - 1) // _L

    def drain_group():
        for r in range(4):
            pltpu.make_async_copy(sdat.at[r], spm.at[sidx.at[r]], sem_s).wait()

    def extract_groups(ng, k, lo, tl, gather_col):
        def per_mg(mg, carry):
            @pl.when(mg >= 2)
            def _():
                drain_group()

            g2 = (mg & 1) * 4
            pk = ml[pl.ds(mg * _L, _L)]
            bm = pk & (_B - 1)
            vrel = ((pk >> 12) - lo) & (_CHUNK - 1)
            padm = (iota + mg * _L) < k
            base = jnp.where(
                padm, tl * _WPT + (bm >> 7) * 1024 + (bm & 127), _DUMP
            )
            for w in range(_W):
                rp = (w // 8) * 32768 + (w % 8) * 128
                data = gather_col(w, vrel)
                row = g2 + w // 8
                sl = pl.ds((w % 8) * _L, _L)
                sdat[row, sl] = data
                sidx[row, sl] = base + rp
            for r in range(4):
                pltpu.async_copy(
                    sdat.at[g2 + r], spm.at[sidx.at[g2 + r]], sem_s
                )
            return carry

        lax.fori_loop(0, ng, per_mg, 0)

        @pl.when(ng >= 2)
        def _():
            drain_group()

        @pl.when(ng >= 1)
        def _():
            drain_group()

    def phase(t0, ntp):
        def process(cid, size):
            lo = cid * _CHUNK

            def l2_g(gi, cnt):
                pk = lvl1[pl.ds(gi * _L, _L)]
                m = (pk >> (12 + _CSH)) == cid
                if size != _CHUNK:
                    m = m & ((pk >> 12) < _VMAIN)
                plsc.store_compressed(ml_dst(ml, cnt), pk, mask=m)
                pc = plsc.all_reduce_population_count(m)
                return cnt + pc[0]

            k = lax.fori_loop(0, ng1, l2_g, 0)
            ng = (k + _L - 1) // _L

            def fire(tl):
                t = c * _TPC + t0 + tl
                pltpu.async_copy(
                    tab_hbm.at[t, :, pl.ds(lo, size)],
                    blk.at[tl & 1, :, pl.ds(0, size)],
                    sem_b,
                )

            def wait_blk(tl):
                t = c * _TPC + t0 + tl
                pltpu.make_async_copy(
                    tab_hbm.at[t, :, pl.ds(lo, size)],
                    blk.at[tl & 1, :, pl.ds(0, size)],
                    sem_b,
                ).wait()

            fire(0)

            def per_table(tl, carry):
                @pl.when(tl + 1 < ntp)
                def _():
                    fire(tl + 1)

                wait_blk(tl)
                slot = tl & 1

                def gather_col(w, vrel):
                    sv = jnp.full((_L,), slot, jnp.int32)
                    wv = jnp.full((_L,), w, jnp.int32)
                    return plsc.load_gather(blk, [sv, wv, vrel])

                extract_groups(ng, k, lo, tl, gather_col)
                return carry

            lax.fori_loop(0, ntp, per_table, 0)

        def chunk_iter(i, carry):
            cid = s + i * _NS

            @pl.when(cid < _NCH - 1)
            def _():
                process(cid, _CHUNK)

            @pl.when(cid == _NCH - 1)
            def _():
                process(cid, _TAILC)

            return carry

        lax.fori_loop(0, -(-_NCH // _NS), chunk_iter, 0)

        @pl.when(s == _NS - 1)
        def _():
            def t_outer(si, cnt):
                pltpu.sync_copy(x_hbm.at[pl.ds(si * 1024, 1024)], xbuf)

                def t_g(gi, cnt2):
                    xg = xbuf[pl.ds(gi * _L, _L)]
                    m = xg >= _VMAIN
                    plsc.store_compressed(
                        ml_dst(ml, cnt2),
                        (xg << 12) | (iota + si * 1024 + gi * _L),
                        mask=m,
                    )
                    pc = plsc.all_reduce_population_count(m)
                    return cnt2 + pc[0]

                return lax.fori_loop(0, 1024 // _L, t_g, cnt)

            k = lax.fori_loop(0, _B // 1024, t_outer, 0)
            ng = (k + _L - 1) // _L

            def per_table(tl, carry):
                t = c * _TPC + t0 + tl
                pltpu.sync_copy(
                    tail_hbm.at[pl.ds(t * (_VTAIL * _W), _VTAIL * _W)], tlv
                )

                def gather_col(w, vrel):
                    flat = (vrel & (_VTAIL - 1)) * _W + w
                    return plsc.load_gather(tlv, [flat])

                extract_groups(ng, k, _VMAIN, tl, gather_col)
                return carry

            lax.fori_loop(0, ntp, per_table, 0)

        plsc.subcore_barrier()

        per_w = ntp * _WPT // _NS
        off = s * per_w
        pltpu.sync_copy(
            spm.at[pl.ds(off, per_w)],
            out_hbm.at[pl.ds((c * _TPC + t0) * _WPT + off, per_w)],
        )
        plsc.subcore_barrier()

    phase(0, _P0)
    phase(_P0, _TPC - _P0)


@jax.jit
def _lookup(tables, x):
    tab_t = jnp.transpose(tables, (0, 2, 1))
    tail = tables[:, _VMAIN:, :].reshape(_NT * _VTAIL * _W)
    mesh = plsc.VectorSubcoreMesh(core_axis_name="c", subcore_axis_name="s")
    out1d = pl.kernel(
        _body,
        compiler_params=pltpu.CompilerParams(needs_layout_passes=False),
        out_type=jax.ShapeDtypeStruct((_NT * _WPT,), jnp.float32),
        mesh=mesh,
        scratch_types=[
            pltpu.VMEM((1024,), jnp.int32),
            pltpu.VMEM((2, _W, _CHUNK), jnp.float32),
            pltpu.VMEM((_VTAIL * _W,), jnp.float32),
            pltpu.VMEM((_B + _L,), jnp.int32),
            pltpu.VMEM((_B + _L,), jnp.int32),
            pltpu.VMEM((8, 128), jnp.float32),
            pltpu.VMEM((8, 128), jnp.int32),
            pltpu.VMEM_SHARED((_SPW + _PAD,), jnp.float32),
            pltpu.SemaphoreType.DMA,
            pltpu.SemaphoreType.DMA,
        ],
    )(tab_t, x, tail)
    out = out1d.reshape(_NT, 4, 32, 8, 128)
    return out.transpose(0, 2, 4, 1, 3).reshape(_NT, _B, _W)


def kernel(x, tables):
    return _lookup(tables, x.astype(jnp.int32))


# speedup vs baseline: 5.4408x; 1.3227x over previous
import jax
import jax.numpy as jnp
from jax import lax
from jax.experimental import pallas as pl
from jax.experimental.pallas import tpu as pltpu
from jax.experimental.pallas import tpu_sc as plsc

_NT = 26
_V = 100000
_W = 32
_B = 4096

_NC = 2
_NS = 16
_L = 16

_TPC = _NT // _NC
_P0 = 5
_CHUNK = 1024
_CSH = 10
_VMAIN = (_V // 128) * 128
_VTAIL = _V - _VMAIN
_NCH = -(-_VMAIN // _CHUNK)
_TAILC = _VMAIN - (_NCH - 1) * _CHUNK
_WPT = _B * _W
_SPW = _P0 * _WPT
_DUMP = _SPW
_PAD = 3 * 32768 + 7 * 128 + _L + 16
_SENT = 1 << 30


def _body(tab_hbm, x_hbm, tail_hbm, out_hbm, xbuf, blk, tlv, ml, lvl1,
          sdat, sidx, spm, sem_b, sem_s):
    c = lax.axis_index("c")
    s = lax.axis_index("s")
    iota = lax.iota(jnp.int32, _L)

    def l1_outer(si, cnt):
        pltpu.sync_copy(x_hbm.at[pl.ds(si * 1024, 1024)], xbuf)

        def l1_g(gi, cnt2):
            xg = xbuf[pl.ds(gi * _L, _L)]
            m = ((xg >> _CSH) & (_NS - 1)) == s
            plsc.store_compressed(
                ml_dst(lvl1, cnt2), (xg << 12) | (iota + si * 1024 + gi * _L),
                mask=m,
            )
            pc = plsc.all_reduce_population_count(m)
            return cnt2 + pc[0]

        return lax.fori_loop(0, 1024 // _L, l1_g, cnt)

    def ml_dst(ref, cnt):
        return ref.at[pl.ds(cnt, _L)]

    k1 = lax.fori_loop(0, _B // 1024, l1_outer, 0)
    lvl1[pl.ds(k1, _L)] = jnp.full((_L,), _SENT, jnp.int32)
    ng1 = (k1 + _L - 1) // _L

    def drain_group():
        for r in range(4):
            pltpu.make_async_copy(sdat.at[r], spm.at[sidx.at[r]], sem_s).wait()

    def extract_groups(ng, k, lo, tl, gather_col):
        def per_mg(mg, carry):
            @pl.when(mg >= 2)
            def _():
                drain_group()

            g2 = (mg & 1) * 4
            pk = ml[pl.ds(mg * _L, _L)]
            bm = pk & (_B - 1)
            vrel = ((pk >> 12) - lo) & (_CHUNK - 1)
            padm = (iota + mg * _L) < k
            base = jnp.where(
                padm, tl * _WPT + (bm >> 7) * 1024 + (bm & 127), _DUMP
            )
            for w in range(_W):
                rp = (w // 8) * 32768 + (w % 8) * 128
                data = gather_col(w, vrel)
                row = g2 + w // 8
                sl = pl.ds((w % 8) * _L, _L)
                sdat[row, sl] = data
                sidx[row, sl] = base + rp
            for r in range(4):
                pltpu.async_copy(
                    sdat.at[g2 + r], spm.at[sidx.at[g2 + r]], sem_s
                )
            return carry

        lax.fori_loop(0, ng, per_mg, 0)

        @pl.when(ng >= 2)
        def _():
            drain_group()

        @pl.when(ng >= 1)
        def _():
            drain_group()

    def phase(t0, ntp):
        def process(cid, size):
            lo = cid * _CHUNK

            def l2_g(gi, cnt):
                pk = lvl1[pl.ds(gi * _L, _L)]
                m = (pk >> (12 + _CSH)) == cid
                if size != _CHUNK:
                    m = m & ((pk >> 12) < _VMAIN)
                plsc.store_compressed(ml_dst(ml, cnt), pk, mask=m)
                pc = plsc.all_reduce_population_count(m)
                return cnt + pc[0]

            k = lax.fori_loop(0, ng1, l2_g, 0)
            ng = (k + _L - 1) // _L

            def fire(tl):
                t = c * _TPC + t0 + tl
                pltpu.async_copy(
                    tab_hbm.at[t, :, pl.ds(lo, size)],
                    blk.at[tl & 1, :, pl.ds(0, size)],
                    sem_b,
                )

            def wait_blk(tl):
                t = c * _TPC + t0 + tl
                pltpu.make_async_copy(
                    tab_hbm.at[t, :, pl.ds(lo, size)],
                    blk.at[tl & 1, :, pl.ds(0, size)],
                    sem_b,
                ).wait()

            fire(0)

            def per_table(tl, carry):
                @pl.when(tl + 1 < ntp)
                def _():
                    fire(tl + 1)

                wait_blk(tl)
                slot = tl & 1

                def gather_col(w, vrel):
                    sv = jnp.full((_L,), slot, jnp.int32)
                    wv = jnp.full((_L,), w, jnp.int32)
                    return plsc.load_gather(blk, [sv, wv, vrel])

                extract_groups(ng, k, lo, tl, gather_col)
                return carry

            lax.fori_loop(0, ntp, per_table, 0)

        def chunk_iter(i, carry):
            cid = s + i * _NS

            @pl.when(cid < _NCH - 1)
            def _():
                process(cid, _CHUNK)

            @pl.when(cid == _NCH - 1)
            def _():
                process(cid, _TAILC)

            return carry

        lax.fori_loop(0, -(-_NCH // _NS), chunk_iter, 0)

        @pl.when(s == _NS - 1)
        def _():
            def t_outer(si, cnt):
                pltpu.sync_copy(x_hbm.at[pl.ds(si * 1024, 1024)], xbuf)

                def t_g(gi, cnt2):
                    xg = xbuf[pl.ds(gi * _L, _L)]
                    m = xg >= _VMAIN
                    plsc.store_compressed(
                        ml_dst(ml, cnt2),
                        (xg << 12) | (iota + si * 1024 + gi * _L),
                        mask=m,
                    )
                    pc = plsc.all_reduce_population_count(m)
                    return cnt2 + pc[0]

                return lax.fori_loop(0, 1024 // _L, t_g, cnt)

            k = lax.fori_loop(0, _B // 1024, t_outer, 0)
            ng = (k + _L - 1) // _L

            def per_table(tl, carry):
                t = c * _TPC + t0 + tl
                pltpu.sync_copy(
                    tail_hbm.at[pl.ds(t * (_VTAIL * _W), _VTAIL * _W)], tlv
                )

                def gather_col(w, vrel):
                    flat = (vrel & (_VTAIL - 1)) * _W + w
                    return plsc.load_gather(tlv, [flat])

                extract_groups(ng, k, _VMAIN, tl, gather_col)
                return carry

            lax.fori_loop(0, ntp, per_table, 0)

        plsc.subcore_barrier()

        per_w = ntp * _WPT // _NS
        off = s * per_w
        pltpu.sync_copy(
            spm.at[pl.ds(off, per_w)],
            out_hbm.at[pl.ds((c * _TPC + t0) * _WPT + off, per_w)],
        )
        plsc.subcore_barrier()

    phase(0, 5)
    phase(5, 4)
    phase(9, 4)


@jax.jit
def _lookup(tables, x):
    tab_t = jnp.transpose(tables, (0, 2, 1))
    tail = tables[:, _VMAIN:, :].reshape(_NT * _VTAIL * _W)
    mesh = plsc.VectorSubcoreMesh(core_axis_name="c", subcore_axis_name="s")
    out1d = pl.kernel(
        _body,
        compiler_params=pltpu.CompilerParams(needs_layout_passes=False),
        out_type=jax.ShapeDtypeStruct((_NT * _WPT,), jnp.float32),
        mesh=mesh,
        scratch_types=[
            pltpu.VMEM((1024,), jnp.int32),
            pltpu.VMEM((2, _W, _CHUNK), jnp.float32),
            pltpu.VMEM((_VTAIL * _W,), jnp.float32),
            pltpu.VMEM((_B + _L,), jnp.int32),
            pltpu.VMEM((_B + _L,), jnp.int32),
            pltpu.VMEM((8, 128), jnp.float32),
            pltpu.VMEM((8, 128), jnp.int32),
            pltpu.VMEM_SHARED((_SPW + _PAD,), jnp.float32),
            pltpu.SemaphoreType.DMA,
            pltpu.SemaphoreType.DMA,
        ],
    )(tab_t, x, tail)
    out = out1d.reshape(_NT, 4, 32, 8, 128)
    return out.transpose(0, 2, 4, 1, 3).reshape(_NT, _B, _W)


def kernel(x, tables):
    return _lookup(tables, x.astype(jnp.int32))


# speedup vs baseline: 5.6387x; 1.0364x over previous
import jax
import jax.numpy as jnp
from jax import lax
from jax.experimental import pallas as pl
from jax.experimental.pallas import tpu as pltpu
from jax.experimental.pallas import tpu_sc as plsc

_NT = 26
_V = 100000
_W = 32
_B = 4096

_NC = 2
_NS = 16
_L = 16

_TPC = _NT // _NC
_P0 = 5
_CHUNK = 1024
_CSH = 10
_VMAIN = (_V // 128) * 128
_VTAIL = _V - _VMAIN
_NCH = -(-_VMAIN // _CHUNK)
_TAILC = _VMAIN - (_NCH - 1) * _CHUNK
_WPT = _B * _W
_SPW = _P0 * _WPT
_DUMP = _SPW
_PAD = 3 * 32768 + 7 * 128 + _L + 16
_SENT = 1 << 30


def _body(tab_hbm, x_hbm, tail_hbm, out_hbm, xbuf, blk, tlv, ml, lvl1,
          sdat, sidx, spm, sem_b, sem_s):
    c = lax.axis_index("c")
    s = lax.axis_index("s")
    iota = lax.iota(jnp.int32, _L)

    def l1_outer(si, cnt):
        pltpu.sync_copy(x_hbm.at[pl.ds(si * 1024, 1024)], xbuf)

        def l1_g(gi, cnt2):
            xg = xbuf[pl.ds(gi * _L, _L)]
            ch = xg >> _CSH
            m = (ch & (_NS - 1)) == s
            m = m | ((s == 4) & (ch == _NCH - 2)) | ((s == 5) & (ch == _NCH - 1))
            plsc.store_compressed(
                ml_dst(lvl1, cnt2), (xg << 12) | (iota + si * 1024 + gi * _L),
                mask=m,
            )
            pc = plsc.all_reduce_population_count(m)
            return cnt2 + pc[0]

        return lax.fori_loop(0, 1024 // _L, l1_g, cnt)

    def ml_dst(ref, cnt):
        return ref.at[pl.ds(cnt, _L)]

    k1 = lax.fori_loop(0, _B // 1024, l1_outer, 0)
    lvl1[pl.ds(k1, _L)] = jnp.full((_L,), _SENT, jnp.int32)
    ng1 = (k1 + _L - 1) // _L

    def drain_group():
        for r in range(4):
            pltpu.make_async_copy(sdat.at[r], spm.at[sidx.at[r]], sem_s).wait()

    def extract_groups(ng, k, lo, tl, gather_col):
        def per_mg(mg, carry):
            @pl.when(mg >= 2)
            def _():
                drain_group()

            g2 = (mg & 1) * 4
            pk = ml[pl.ds(mg * _L, _L)]
            bm = pk & (_B - 1)
            vrel = ((pk >> 12) - lo) & (_CHUNK - 1)
            padm = (iota + mg * _L) < k
            base = jnp.where(
                padm, tl * _WPT + (bm >> 7) * 1024 + (bm & 127), _DUMP
            )
            for w in range(_W):
                rp = (w // 8) * 32768 + (w % 8) * 128
                data = gather_col(w, vrel)
                row = g2 + w // 8
                sl = pl.ds((w % 8) * _L, _L)
                sdat[row, sl] = data
                sidx[row, sl] = base + rp
            for r in range(4):
                pltpu.async_copy(
                    sdat.at[g2 + r], spm.at[sidx.at[g2 + r]], sem_s
                )
            return carry

        lax.fori_loop(0, ng, per_mg, 0)

        @pl.when(ng >= 2)
        def _():
            drain_group()

        @pl.when(ng >= 1)
        def _():
            drain_group()

    def phase(t0, ntp):
        def process(cid, tl0, tcnt):
            lo = pl.multiple_of(
                jnp.minimum(cid * _CHUNK, _VMAIN - _CHUNK), 128
            )

            def l2_g(gi, cnt):
                pk = lvl1[pl.ds(gi * _L, _L)]
                m = ((pk >> (12 + _CSH)) == cid) & ((pk >> 12) < _VMAIN)
                plsc.store_compressed(ml_dst(ml, cnt), pk, mask=m)
                pc = plsc.all_reduce_population_count(m)
                return cnt + pc[0]

            k = lax.fori_loop(0, ng1, l2_g, 0)
            ng = (k + _L - 1) // _L

            def fire(tl):
                t = c * _TPC + t0 + tl0 + tl
                pltpu.async_copy(
                    tab_hbm.at[t, :, pl.ds(lo, _CHUNK)],
                    blk.at[tl & 1],
                    sem_b,
                )

            def wait_blk(tl):
                t = c * _TPC + t0 + tl0 + tl
                pltpu.make_async_copy(
                    tab_hbm.at[t, :, pl.ds(lo, _CHUNK)],
                    blk.at[tl & 1],
                    sem_b,
                ).wait()

            fire(0)

            def per_table(tl, carry):
                @pl.when(tl + 1 < tcnt)
                def _():
                    fire(tl + 1)

                wait_blk(tl)
                slot = tl & 1

                def gather_col(w, vrel):
                    sv = jnp.full((_L,), slot, jnp.int32)
                    wv = jnp.full((_L,), w, jnp.int32)
                    return plsc.load_gather(blk, [sv, wv, vrel])

                extract_groups(ng, k, lo, tl0 + tl, gather_col)
                return carry

            lax.fori_loop(0, tcnt, per_table, 0)

        def chunk_iter(i, carry):
            cid = s + i * _NS
            process(cid, 0, ntp)
            return carry

        lax.fori_loop(0, (_NCH - 2) // _NS, chunk_iter, 0)

        hi = ntp // 2

        @pl.when((s == 0) | (s == 1) | (s == 4) | (s == 5))
        def _():
            cidx = jnp.where((s == 0) | (s == 4), _NCH - 2, _NCH - 1)
            tl0 = jnp.where(s < 4, 0, ntp - hi)
            tcnt = jnp.where(s < 4, ntp - hi, hi)
            process(cidx, tl0, tcnt)

        @pl.when(s == _NS - 1)
        def _():
            def t_outer(si, cnt):
                pltpu.sync_copy(x_hbm.at[pl.ds(si * 1024, 1024)], xbuf)

                def t_g(gi, cnt2):
                    xg = xbuf[pl.ds(gi * _L, _L)]
                    m = xg >= _VMAIN
                    plsc.store_compressed(
                        ml_dst(ml, cnt2),
                        (xg << 12) | (iota + si * 1024 + gi * _L),
                        mask=m,
                    )
                    pc = plsc.all_reduce_population_count(m)
                    return cnt2 + pc[0]

                return lax.fori_loop(0, 1024 // _L, t_g, cnt)

            k = lax.fori_loop(0, _B // 1024, t_outer, 0)
            ng = (k + _L - 1) // _L

            def per_table(tl, carry):
                t = c * _TPC + t0 + tl
                pltpu.sync_copy(
                    tail_hbm.at[pl.ds(t * (_VTAIL * _W), _VTAIL * _W)], tlv
                )

                def gather_col(w, vrel):
                    flat = (vrel & (_VTAIL - 1)) * _W + w
                    return plsc.load_gather(tlv, [flat])

                extract_groups(ng, k, _VMAIN, tl, gather_col)
                return carry

            lax.fori_loop(0, ntp, per_table, 0)

        plsc.subcore_barrier()

        per_w = ntp * _WPT // _NS
        off = s * per_w
        pltpu.sync_copy(
            spm.at[pl.ds(off, per_w)],
            out_hbm.at[pl.ds((c * _TPC + t0) * _WPT + off, per_w)],
        )
        plsc.subcore_barrier()

    phase(0, 5)
    phase(5, 4)
    phase(9, 4)


@jax.jit
def _lookup(tables, x):
    tab_t = jnp.transpose(tables, (0, 2, 1))
    tail = tables[:, _VMAIN:, :].reshape(_NT * _VTAIL * _W)
    mesh = plsc.VectorSubcoreMesh(core_axis_name="c", subcore_axis_name="s")
    out1d = pl.kernel(
        _body,
        compiler_params=pltpu.CompilerParams(needs_layout_passes=False),
        out_type=jax.ShapeDtypeStruct((_NT * _WPT,), jnp.float32),
        mesh=mesh,
        scratch_types=[
            pltpu.VMEM((1024,), jnp.int32),
            pltpu.VMEM((2, _W, _CHUNK), jnp.float32),
            pltpu.VMEM((_VTAIL * _W,), jnp.float32),
            pltpu.VMEM((_B + _L,), jnp.int32),
            pltpu.VMEM((_B + _L,), jnp.int32),
            pltpu.VMEM((8, 128), jnp.float32),
            pltpu.VMEM((8, 128), jnp.int32),
            pltpu.VMEM_SHARED((_SPW + _PAD,), jnp.float32),
            pltpu.SemaphoreType.DMA,
            pltpu.SemaphoreType.DMA,
        ],
    )(tab_t, x, tail)
    out = out1d.reshape(_NT, 4, 32, 8, 128)
    return out.transpose(0, 2, 4, 1, 3).reshape(_NT, _B, _W)


def kernel(x, tables):
    return _lookup(tables, x.astype(jnp.int32))
